# baseline (device time: 31448 ns/iter reference)
import jax
import jax.numpy as jnp
from jax import lax
from jax.experimental import pallas as pl
from jax.experimental.pallas import tpu as pltpu

N_DEV = 8
N_LAYERS = 3
B, D = 64, 512
R = B // N_DEV
NC = 2
W = D // NC
N_LC = 2 * NC

A_MASKS = (1, 3, 2)
B_MASK = 4
ALL_MASKS = tuple(range(1, N_DEV))


def kernel(x, Win0, Wout0, Win1, Wout1, Win2, Wout2):
    def body(
        x_ref,
        win0_ref,
        wout0_ref,
        win1_ref,
        wout1_ref,
        win2_ref,
        wout2_ref,
        out_ref,
        send_ref,
        recv_ref,
        rs_stage,
        rs_recv,
        win_buf,
        wout_buf,
        send_sems,
        recv_sems,
        rs_send_sems,
        rs_recv_sems,
        w_sems,
    ):
        my = lax.axis_index("i")
        wins_hbm = [win0_ref, win1_ref, win2_ref]
        wouts_hbm = [wout0_ref, wout1_ref, wout2_ref]

        def w_copies(k):
            s = k % 2
            return (
                pltpu.make_async_copy(wins_hbm[k], win_buf.at[s], w_sems.at[k, 0]),
                pltpu.make_async_copy(wouts_hbm[k], wout_buf.at[s], w_sems.at[k, 1]),
            )

        for cp in w_copies(0) + w_copies(1):
            cp.start()

        def wait_w(k):
            for cp in w_copies(k):
                cp.wait()

        wins = [win_buf.at[k % 2] for k in range(N_LAYERS)]
        wouts = [wout_buf.at[k % 2] for k in range(N_LAYERS)]

        barrier_sem = pltpu.get_barrier_semaphore()
        for m in ALL_MASKS:
            pl.semaphore_signal(
                barrier_sem,
                inc=1,
                device_id=(my ^ m,),
                device_id_type=pl.DeviceIdType.MESH,
            )
        pl.semaphore_wait(barrier_sem, len(ALL_MASKS))

        def make_rdma(lc, phase_slot, mask, j):
            return pltpu.make_async_remote_copy(
                src_ref=send_ref.at[lc, phase_slot],
                dst_ref=recv_ref.at[lc, j],
                send_sem=send_sems.at[lc, j],
                recv_sem=recv_sems.at[lc, j],
                device_id=(my ^ mask,),
                device_id_type=pl.DeviceIdType.MESH,
            )

        def start_a(lc, val):
            send_ref[lc, 0, :, :] = val.astype(jnp.bfloat16)
            rdmas = [make_rdma(lc, 0, m, j) for j, m in enumerate(A_MASKS)]
            for r in rdmas:
                r.start()
            return rdmas

        def start_b(lc, val):
            send_ref[lc, 1, :, :] = val.astype(jnp.bfloat16)
            r = make_rdma(lc, 1, B_MASK, 3)
            r.start()
            return [r]

        wait_w(0)
        h = jnp.maximum(
            jnp.dot(x_ref[:, :], wins[0][:, :], preferred_element_type=jnp.float32),
            0.0,
        )
        for layer in range(2):
            acc = [None] * NC
            rdmas = {}
            for c in range(NC):
                lc = layer * NC + c
                acc[c] = jnp.dot(
                    h,
                    wouts[layer][:, c * W : (c + 1) * W],
                    preferred_element_type=jnp.float32,
                )
                rdmas[c] = start_a(lc, acc[c])
            if layer == 0:
                for cp in w_copies(2):
                    cp.start()
            hacc = None
            for c in range(NC):
                lc = layer * NC + c
                for r in rdmas[c]:
                    r.wait()
                acc[c] = acc[c] + (
                    recv_ref[lc, 0, :, :]
                    + recv_ref[lc, 1, :, :]
                    + recv_ref[lc, 2, :, :]
                ).astype(jnp.float32)
                rdmas[c] = start_b(lc, acc[c])
            wait_w(layer + 1)
            for c in range(NC):
                lc = layer * NC + c
                for r in rdmas[c]:
                    r.wait()
                acc[c] = acc[c] + recv_ref[lc, 3, :, :].astype(jnp.float32)
                contrib = jnp.dot(
                    acc[c],
                    wins[layer + 1][c * W : (c + 1) * W, :],
                    preferred_element_type=jnp.float32,
                )
                hacc = contrib if hacc is None else hacc + contrib
            h = jnp.maximum(hacc, 0.0)

        for c in range(NC):
            rs_stage[:, c * W : (c + 1) * W] = jnp.dot(
                h,
                wouts[2][:, c * W : (c + 1) * W],
                preferred_element_type=jnp.float32,
            ).astype(jnp.bfloat16)
        rs_rdmas = []
        for m in ALL_MASKS:
            d = my ^ m
            r = pltpu.make_async_remote_copy(
                src_ref=rs_stage.at[pl.ds(d * R, R), :],
                dst_ref=rs_recv.at[m - 1],
                send_sem=rs_send_sems.at[m - 1],
                recv_sem=rs_recv_sems.at[m - 1],
                device_id=(d,),
                device_id_type=pl.DeviceIdType.MESH,
            )
            r.start()
            rs_rdmas.append(r)
        out = rs_stage[pl.ds(my * R, R), :].astype(jnp.float32)
        for j, r in enumerate(rs_rdmas):
            r.wait()
            out = out + rs_recv[j, :, :].astype(jnp.float32)
        out_ref[:, :] = out

    return pl.pallas_call(
        body,
        out_shape=jax.ShapeDtypeStruct((R, D), jnp.float32),
        in_specs=[pl.BlockSpec(memory_space=pltpu.VMEM)]
        + [pl.BlockSpec(memory_space=pltpu.MemorySpace.HBM)] * 6,
        out_specs=pl.BlockSpec(memory_space=pltpu.VMEM),
        scratch_shapes=[
            pltpu.VMEM((N_LC, 2, B, W), jnp.bfloat16),
            pltpu.VMEM((N_LC, 4, B, W), jnp.bfloat16),
            pltpu.VMEM((B, D), jnp.bfloat16),
            pltpu.VMEM((N_DEV - 1, R, D), jnp.bfloat16),
            pltpu.VMEM((2, 512, 1024), jnp.float32),
            pltpu.VMEM((2, 1024, 512), jnp.float32),
            pltpu.SemaphoreType.DMA((N_LC, 4)),
            pltpu.SemaphoreType.DMA((N_LC, 4)),
            pltpu.SemaphoreType.DMA((N_DEV - 1,)),
            pltpu.SemaphoreType.DMA((N_DEV - 1,)),
            pltpu.SemaphoreType.DMA((N_LAYERS, 2)),
        ],
        compiler_params=pltpu.CompilerParams(collective_id=0),
    )(x, Win0, Wout0, Win1, Wout1, Win2, Wout2)


# device time: 30587 ns/iter; 1.0281x vs baseline; 1.0281x over previous
import jax
import jax.numpy as jnp
from jax import lax
from jax.experimental import pallas as pl
from jax.experimental.pallas import tpu as pltpu

N_DEV = 8
N_LAYERS = 3
B, D = 64, 512
R = B // N_DEV
NC = 4
W = D // NC
N_LC = 2 * NC

A_MASKS = (2, 1, 3)
B_MASK = 4
ALL_MASKS = tuple(range(1, N_DEV))
RS_MASKS = (6, 2, 5, 7, 1, 3, 4)


def kernel(x, Win0, Wout0, Win1, Wout1, Win2, Wout2):
    def body(
        x_ref,
        win0_ref,
        wout0_ref,
        win1_ref,
        wout1_ref,
        win2_ref,
        wout2_ref,
        out_ref,
        send_ref,
        recv_ref,
        rs_stage,
        rs_recv,
        win_buf,
        wout_buf,
        send_sems,
        recv_sems,
        rs_send_sems,
        rs_recv_sems,
        w_sems,
    ):
        my = lax.axis_index("i")
        wins_hbm = [win0_ref, win1_ref, win2_ref]
        wouts_hbm = [wout0_ref, wout1_ref, wout2_ref]

        def w_copies(k):
            s = k % 2
            return (
                pltpu.make_async_copy(wins_hbm[k], win_buf.at[s], w_sems.at[k, 0]),
                pltpu.make_async_copy(wouts_hbm[k], wout_buf.at[s], w_sems.at[k, 1]),
            )

        for cp in w_copies(0) + w_copies(1):
            cp.start()

        def wait_w(k):
            for cp in w_copies(k):
                cp.wait()

        wins = [win_buf.at[k % 2] for k in range(N_LAYERS)]
        wouts = [wout_buf.at[k % 2] for k in range(N_LAYERS)]

        barrier_sem = pltpu.get_barrier_semaphore()
        for m in ALL_MASKS:
            pl.semaphore_signal(
                barrier_sem,
                inc=1,
                device_id=(my ^ m,),
                device_id_type=pl.DeviceIdType.MESH,
            )

        def make_rdma(lc, phase_slot, mask, j):
            return pltpu.make_async_remote_copy(
                src_ref=send_ref.at[lc, phase_slot],
                dst_ref=recv_ref.at[lc, j],
                send_sem=send_sems.at[lc, j],
                recv_sem=recv_sems.at[lc, j],
                device_id=(my ^ mask,),
                device_id_type=pl.DeviceIdType.MESH,
            )

        def start_a(lc, val):
            send_ref[lc, 0, :, :] = val.astype(jnp.bfloat16)
            rdmas = [make_rdma(lc, 0, m, j) for j, m in enumerate(A_MASKS)]
            for r in rdmas:
                r.start()
            return rdmas

        def start_b(lc, val):
            send_ref[lc, 1, :, :] = val.astype(jnp.bfloat16)
            r = make_rdma(lc, 1, B_MASK, 3)
            r.start()
            return [r]

        wait_w(0)
        h = jnp.maximum(
            jnp.dot(x_ref[:, :], wins[0][:, :], preferred_element_type=jnp.float32),
            0.0,
        )
        pl.semaphore_wait(barrier_sem, len(ALL_MASKS))
        for layer in range(2):
            acc = [None] * NC
            rdmas = {}
            for c in range(NC):
                lc = layer * NC + c
                acc[c] = jnp.dot(
                    h,
                    wouts[layer][:, c * W : (c + 1) * W],
                    preferred_element_type=jnp.float32,
                )
                rdmas[c] = start_a(lc, acc[c])
            if layer == 0:
                for cp in w_copies(2):
                    cp.start()
            hacc = None
            for c in range(NC):
                lc = layer * NC + c
                for r in rdmas[c]:
                    r.wait()
                acc[c] = acc[c] + (
                    recv_ref[lc, 0, :, :]
                    + recv_ref[lc, 1, :, :]
                    + recv_ref[lc, 2, :, :]
                ).astype(jnp.float32)
                rdmas[c] = start_b(lc, acc[c])
            wait_w(layer + 1)
            for c in range(NC):
                lc = layer * NC + c
                for r in rdmas[c]:
                    r.wait()
                acc[c] = acc[c] + recv_ref[lc, 3, :, :].astype(jnp.float32)
                contrib = jnp.dot(
                    acc[c],
                    wins[layer + 1][c * W : (c + 1) * W, :],
                    preferred_element_type=jnp.float32,
                )
                hacc = contrib if hacc is None else hacc + contrib
            h = jnp.maximum(hacc, 0.0)

        for c in range(NC):
            rs_stage[:, c * W : (c + 1) * W] = jnp.dot(
                h,
                wouts[2][:, c * W : (c + 1) * W],
                preferred_element_type=jnp.float32,
            ).astype(jnp.bfloat16)
        rs_rdmas = []
        for m in RS_MASKS:
            d = my ^ m
            r = pltpu.make_async_remote_copy(
                src_ref=rs_stage.at[pl.ds(d * R, R), :],
                dst_ref=rs_recv.at[m - 1],
                send_sem=rs_send_sems.at[m - 1],
                recv_sem=rs_recv_sems.at[m - 1],
                device_id=(d,),
                device_id_type=pl.DeviceIdType.MESH,
            )
            r.start()
            rs_rdmas.append(r)
        out = rs_stage[pl.ds(my * R, R), :].astype(jnp.float32)
        for m, r in reversed(list(zip(RS_MASKS, rs_rdmas))):
            r.wait()
            out = out + rs_recv[m - 1, :, :].astype(jnp.float32)
        out_ref[:, :] = out

    return pl.pallas_call(
        body,
        out_shape=jax.ShapeDtypeStruct((R, D), jnp.float32),
        in_specs=[pl.BlockSpec(memory_space=pltpu.VMEM)]
        + [pl.BlockSpec(memory_space=pltpu.MemorySpace.HBM)] * 6,
        out_specs=pl.BlockSpec(memory_space=pltpu.VMEM),
        scratch_shapes=[
            pltpu.VMEM((N_LC, 2, B, W), jnp.bfloat16),
            pltpu.VMEM((N_LC, 4, B, W), jnp.bfloat16),
            pltpu.VMEM((B, D), jnp.bfloat16),
            pltpu.VMEM((N_DEV - 1, R, D), jnp.bfloat16),
            pltpu.VMEM((2, 512, 1024), jnp.float32),
            pltpu.VMEM((2, 1024, 512), jnp.float32),
            pltpu.SemaphoreType.DMA((N_LC, 4)),
            pltpu.SemaphoreType.DMA((N_LC, 4)),
            pltpu.SemaphoreType.DMA((N_DEV - 1,)),
            pltpu.SemaphoreType.DMA((N_DEV - 1,)),
            pltpu.SemaphoreType.DMA((N_LAYERS, 2)),
        ],
        compiler_params=pltpu.CompilerParams(collective_id=0),
    )(x, Win0, Wout0, Win1, Wout1, Win2, Wout2)


# device time: 29206 ns/iter; 1.0768x vs baseline; 1.0473x over previous
import jax
import jax.numpy as jnp
from jax import lax
from jax.experimental import pallas as pl
from jax.experimental.pallas import tpu as pltpu

N_DEV = 8
N_LAYERS = 3
B, D, H = 64, 512, 1024
R = B // N_DEV
NR = 4
RB = B // NR
N_LB = 2 * NR

AR_COMM = True
RS_COMM = True
COMM = AR_COMM or RS_COMM

A_MASKS = (2, 1, 3)
B_MASK = 4
ALL_MASKS = tuple(range(1, N_DEV))
RS_MASKS = (6, 2, 5, 7, 1, 3, 4)


def kernel(x, Win0, Wout0, Win1, Wout1, Win2, Wout2):
    def body(
        x_ref,
        win0_ref,
        wout0_ref,
        win1_ref,
        wout1_ref,
        win2_ref,
        wout2_ref,
        out_ref,
        send_ref,
        recv_ref,
        rs_stage,
        rs_recv,
        win_buf,
        wout_buf,
        send_sems,
        recv_sems,
        rs_send_sems,
        rs_recv_sems,
        w_sems,
    ):
        my = lax.axis_index("i")
        wins_hbm = [win0_ref, win1_ref, win2_ref]
        wouts_hbm = [wout0_ref, wout1_ref, wout2_ref]

        def w_copies(k):
            s = k % 2
            return (
                pltpu.make_async_copy(wins_hbm[k], win_buf.at[s], w_sems.at[k, 0]),
                pltpu.make_async_copy(wouts_hbm[k], wout_buf.at[s], w_sems.at[k, 1]),
            )

        for cp in w_copies(0) + w_copies(1):
            cp.start()

        def wait_w(k):
            for cp in w_copies(k):
                cp.wait()

        wins = [win_buf.at[k % 2] for k in range(N_LAYERS)]
        wouts = [wout_buf.at[k % 2] for k in range(N_LAYERS)]

        barrier_sem = pltpu.get_barrier_semaphore() if COMM else None
        for m in ALL_MASKS if COMM else ():
            pl.semaphore_signal(
                barrier_sem,
                inc=1,
                device_id=(my ^ m,),
                device_id_type=pl.DeviceIdType.MESH,
            )

        def make_rdma(lb, phase_slot, mask, j):
            return pltpu.make_async_remote_copy(
                src_ref=send_ref.at[lb, phase_slot],
                dst_ref=recv_ref.at[lb, j],
                send_sem=send_sems.at[lb, j],
                recv_sem=recv_sems.at[lb, j],
                device_id=(my ^ mask,),
                device_id_type=pl.DeviceIdType.MESH,
            )

        def start_a(lb, val):
            send_ref[lb, 0, :, :] = val.astype(jnp.bfloat16)
            if not AR_COMM:
                return []
            rdmas = [make_rdma(lb, 0, m, j) for j, m in enumerate(A_MASKS)]
            for r in rdmas:
                r.start()
            return rdmas

        def start_b(lb, val):
            send_ref[lb, 1, :, :] = val.astype(jnp.bfloat16)
            if not AR_COMM:
                return []
            r = make_rdma(lb, 1, B_MASK, 3)
            r.start()
            return [r]

        def quad_add(lb, val):
            return val + (
                recv_ref[lb, 0, :, :]
                + recv_ref[lb, 1, :, :]
                + recv_ref[lb, 2, :, :]
            ).astype(jnp.float32)

        def z_add(lb, val):
            return val + recv_ref[lb, 3, :, :].astype(jnp.float32)

        wait_w(0)
        acc = [None] * NR
        rdmas = {}
        barrier_done = False
        for b in range(NR):
            hb = jnp.maximum(
                jnp.dot(
                    x_ref[b * RB : (b + 1) * RB, :],
                    wins[0][:, :],
                    preferred_element_type=jnp.float32,
                ),
                0.0,
            )
            acc[b] = jnp.dot(
                hb, wouts[0][:, :], preferred_element_type=jnp.float32
            )
            if not barrier_done and COMM:
                pl.semaphore_wait(barrier_sem, len(ALL_MASKS))
                barrier_done = True
            rdmas[b] = start_a(b, acc[b])
        for cp in w_copies(2):
            cp.start()

        for layer in range(2):
            for b in range(NR):
                lb = layer * NR + b
                for r in rdmas[b]:
                    r.wait()
                acc[b] = quad_add(lb, acc[b])
                rdmas[b] = start_b(lb, acc[b])
            wait_w(layer + 1)
            for b in range(NR):
                lb = layer * NR + b
                for r in rdmas[b]:
                    r.wait()
                xb = z_add(lb, acc[b])
                hb = jnp.maximum(
                    jnp.dot(
                        xb,
                        wins[layer + 1][:, :],
                        preferred_element_type=jnp.float32,
                    ),
                    0.0,
                )
                pb = jnp.dot(
                    hb, wouts[layer + 1][:, :], preferred_element_type=jnp.float32
                )
                if layer < 1:
                    acc[b] = pb
                    rdmas[b] = start_a((layer + 1) * NR + b, pb)
                else:
                    rs_stage[b * RB : (b + 1) * RB, :] = pb.astype(jnp.bfloat16)

        rs_rdmas = []
        for m in RS_MASKS if RS_COMM else ():
            d = my ^ m
            r = pltpu.make_async_remote_copy(
                src_ref=rs_stage.at[pl.ds(d * R, R), :],
                dst_ref=rs_recv.at[m - 1],
                send_sem=rs_send_sems.at[m - 1],
                recv_sem=rs_recv_sems.at[m - 1],
                device_id=(d,),
                device_id_type=pl.DeviceIdType.MESH,
            )
            r.start()
            rs_rdmas.append(r)
        out = rs_stage[pl.ds(my * R, R), :].astype(jnp.float32)
        for m, r in reversed(list(zip(RS_MASKS, rs_rdmas))):
            r.wait()
            out = out + rs_recv[m - 1, :, :].astype(jnp.float32)
        out_ref[:, :] = out

    return pl.pallas_call(
        body,
        out_shape=jax.ShapeDtypeStruct((R, D), jnp.float32),
        in_specs=[pl.BlockSpec(memory_space=pltpu.VMEM)]
        + [pl.BlockSpec(memory_space=pltpu.MemorySpace.HBM)] * 6,
        out_specs=pl.BlockSpec(memory_space=pltpu.VMEM),
        scratch_shapes=[
            pltpu.VMEM((N_LB, 2, RB, D), jnp.bfloat16),
            pltpu.VMEM((N_LB, 4, RB, D), jnp.bfloat16),
            pltpu.VMEM((B, D), jnp.bfloat16),
            pltpu.VMEM((N_DEV - 1, R, D), jnp.bfloat16),
            pltpu.VMEM((2, 512, 1024), jnp.float32),
            pltpu.VMEM((2, 1024, 512), jnp.float32),
            pltpu.SemaphoreType.DMA((N_LB, 4)),
            pltpu.SemaphoreType.DMA((N_LB, 4)),
            pltpu.SemaphoreType.DMA((N_DEV - 1,)),
            pltpu.SemaphoreType.DMA((N_DEV - 1,)),
            pltpu.SemaphoreType.DMA((N_LAYERS, 2)),
        ],
        compiler_params=pltpu.CompilerParams(collective_id=0 if COMM else None),
    )(x, Win0, Wout0, Win1, Wout1, Win2, Wout2)
